# baseline (speedup 1.0000x reference)
import jax
import jax.numpy as jnp
from jax.experimental import pallas as pl

_RATIO = 16


def _body(mask_ref, w_ref, emb_ref, gmask_ref):
    B, Sg, D = emb_ref.shape
    w0 = w_ref[0, :]
    w1 = w_ref[1, :]
    emb_ref[...] = jnp.broadcast_to(w0[None, None, :], (B, Sg, D))
    emb_ref[:, 0, :] = jnp.broadcast_to(w1[None, :], (B, D))

    m = mask_ref[...]
    s = 1
    while s < _RATIO:
        m = m & jnp.concatenate([m[:, s:], m[:, :s]], axis=1)
        s *= 2
    gmask_ref[...] = m


def kernel(token_ids, padding_mask, embeds_weight):
    B, Sl = padding_mask.shape
    Sg = Sl // _RATIO
    D = embeds_weight.shape[1]
    mask2 = padding_mask.astype(jnp.int32)
    emb, gmask = pl.pallas_call(
        _body,
        out_shape=(
            jax.ShapeDtypeStruct((B, Sg, D), embeds_weight.dtype),
            jax.ShapeDtypeStruct((B, Sl), jnp.int32),
        ),
    )(mask2, embeds_weight)
    return (emb, gmask[:, ::_RATIO] != 0)


# speedup vs baseline: 1.0566x; 1.0566x over previous
import jax
import jax.numpy as jnp
from jax.experimental import pallas as pl

_RATIO = 16


def _body(mask_ref, w_ref, emb_ref, gmask_ref):
    B, Sg, D = emb_ref.shape
    w0 = w_ref[0, :]
    w1 = w_ref[1, :]
    emb_ref[...] = jnp.broadcast_to(w0[None, None, :], (B, Sg, D))
    emb_ref[:, 0, :] = jnp.broadcast_to(w1[None, :], (B, D))

    Bm, Sl = mask_ref.shape
    L = 128
    G = L // _RATIO
    mf = mask_ref[...].astype(jnp.float32).reshape(Bm * Sl // L, L)
    sel = (jax.lax.broadcasted_iota(jnp.int32, (L, G), 0) // _RATIO
           == jax.lax.broadcasted_iota(jnp.int32, (L, G), 1)
           ).astype(jnp.float32)
    s = jax.lax.dot_general(mf, sel, (((1,), (0,)), ((), ())),
                            preferred_element_type=jnp.float32)
    gmask_ref[...] = jnp.where(s == float(_RATIO), 1, 0).astype(jnp.int32)


def kernel(token_ids, padding_mask, embeds_weight):
    B, Sl = padding_mask.shape
    Sg = Sl // _RATIO
    D = embeds_weight.shape[1]
    mask2 = padding_mask.astype(jnp.int8)
    emb, gmask = pl.pallas_call(
        _body,
        out_shape=(
            jax.ShapeDtypeStruct((B, Sg, D), embeds_weight.dtype),
            jax.ShapeDtypeStruct((B * Sl // 128, 128 // _RATIO), jnp.int32),
        ),
    )(mask2, embeds_weight)
    return (emb, gmask.reshape(B, Sg) != 0)


# speedup vs baseline: 1.0576x; 1.0009x over previous
import jax
import jax.numpy as jnp
from jax.experimental import pallas as pl

_RATIO = 16


def _body(mask_ref, w_ref, emb_ref, gmask_ref):
    B, Sg, D = emb_ref.shape
    w0 = w_ref[0, :]
    w1 = w_ref[1, :]
    emb_ref[...] = jnp.broadcast_to(w0[None, None, :], (B, Sg, D))
    emb_ref[:, 0, :] = jnp.broadcast_to(w1[None, :], (B, D))

    Bm, Sl = mask_ref.shape
    L = 128
    G = L // _RATIO
    mf = mask_ref[...].astype(jnp.float32).reshape(Bm * Sl // L, L)
    sel = (jax.lax.broadcasted_iota(jnp.int32, (L, G), 0) // _RATIO
           == jax.lax.broadcasted_iota(jnp.int32, (L, G), 1)
           ).astype(jnp.float32)
    s = jax.lax.dot_general(mf, sel, (((1,), (0,)), ((), ())),
                            preferred_element_type=jnp.float32)
    gmask_ref[...] = jnp.where(s == float(_RATIO), 1, 0).astype(jnp.int32)


def kernel(token_ids, padding_mask, embeds_weight):
    B, Sl = padding_mask.shape
    Sg = Sl // _RATIO
    D = embeds_weight.shape[1]
    mask2 = jnp.where(padding_mask, jnp.int8(1), jnp.int8(0))
    emb, gmask = pl.pallas_call(
        _body,
        out_shape=(
            jax.ShapeDtypeStruct((B, Sg, D), embeds_weight.dtype),
            jax.ShapeDtypeStruct((B * Sl // 128, 128 // _RATIO), jnp.int32),
        ),
    )(mask2, embeds_weight)
    return (emb, gmask.reshape(B, Sg) != 0)


# speedup vs baseline: 1.0761x; 1.0175x over previous
import jax
import jax.numpy as jnp
from jax.experimental import pallas as pl

_RATIO = 16


def _body(mask_ref, w_ref, emb_ref, gmask_ref):
    B, Sg, D = emb_ref.shape
    w0 = w_ref[0, :]
    w1 = w_ref[1, :]
    emb_ref[...] = jnp.broadcast_to(w0[None, None, :], (B, Sg, D))
    emb_ref[:, 0, :] = jnp.broadcast_to(w1[None, :], (B, D))

    Bm, Sl = mask_ref.shape
    L = 128
    G = L // _RATIO
    mf = mask_ref[...].astype(jnp.float32).reshape(Bm * Sl // L, L)
    sel = (jax.lax.broadcasted_iota(jnp.int32, (L, G), 0) // _RATIO
           == jax.lax.broadcasted_iota(jnp.int32, (L, G), 1)
           ).astype(jnp.float32)
    s = jax.lax.dot_general(mf, sel, (((1,), (0,)), ((), ())),
                            preferred_element_type=jnp.float32)
    gmask_ref[...] = jnp.where(s == float(_RATIO), 1, 0).astype(jnp.int8)


def kernel(token_ids, padding_mask, embeds_weight):
    B, Sl = padding_mask.shape
    Sg = Sl // _RATIO
    D = embeds_weight.shape[1]
    mask2 = padding_mask.view(jnp.int8)
    emb, gmask = pl.pallas_call(
        _body,
        out_shape=(
            jax.ShapeDtypeStruct((B, Sg, D), embeds_weight.dtype),
            jax.ShapeDtypeStruct((B * Sl // 128, 128 // _RATIO), jnp.int8),
        ),
    )(mask2, embeds_weight)
    return (emb, gmask.reshape(B, Sg).view(jnp.bool_))
